# baseline (speedup 1.0000x reference)
import functools

import jax
import jax.numpy as jnp
from jax import lax
from jax.experimental import pallas as pl
from jax.experimental.pallas import tpu as pltpu

MARGIN_ = 1.0
EPS_ = 1e-6
CB_ = 512


def _fused_body(inputs_ref, labels_ref, target_ref, out_ref, min_acc, pos_acc,
                *, n_valid):
    i = pl.program_id(0)
    nsteps = pl.num_programs(0)

    a = inputs_ref[...]
    t = target_ref[...] - EPS_

    @pl.when(i == 0)
    def _init():
        min_acc[...] = jnp.full_like(min_acc, jnp.inf)
        pos_acc[...] = jnp.zeros_like(pos_acc)

    dots = lax.dot_general(a, t, (((1,), (1,)), ((), ())),
                           preferred_element_type=jnp.float32)
    ones = jnp.ones((1, a.shape[1]), jnp.float32)
    t_sq = lax.dot_general(ones, t * t, (((1,), (1,)), ((), ())),
                           preferred_element_type=jnp.float32)
    s = t_sq - 2.0 * dots

    cols = i * CB_ + lax.broadcasted_iota(jnp.int32, s.shape, 1)
    is_own = cols == labels_ref[...]
    invalid = is_own | (cols >= n_valid)
    s_min = jnp.where(invalid, jnp.inf, s)
    s_pos = jnp.where(is_own, s, 0.0)

    m = min_acc[...]
    p = pos_acc[...]
    for k in range(CB_ // 128):
        m = jnp.minimum(m, s_min[:, k * 128:(k + 1) * 128])
        p = p + s_pos[:, k * 128:(k + 1) * 128]
    min_acc[...] = m
    pos_acc[...] = p

    @pl.when(i == nsteps - 1)
    def _finish():
        a_sq = jnp.sum(a * a, axis=1, keepdims=True)
        d_an = jnp.sqrt(jnp.clip(
            a_sq + jnp.min(min_acc[...], axis=1, keepdims=True), 1e-12))
        d_ap = jnp.sqrt(jnp.clip(
            a_sq + jnp.sum(pos_acc[...], axis=1, keepdims=True), 1e-12))
        per = jnp.maximum(d_ap - d_an + MARGIN_, 0.0)
        out_ref[0, 0] = jnp.sum(per) / a.shape[0]


def kernel(inputs, labels, target):
    B, D = inputs.shape
    C = target.shape[0]
    nblocks = (C + CB_ - 1) // CB_
    cpad = nblocks * CB_
    tpad = jnp.pad(target, ((0, cpad - C), (0, 0)))
    labels2 = labels.reshape(B, 1)
    out = pl.pallas_call(
        functools.partial(_fused_body, n_valid=C),
        grid=(nblocks,),
        in_specs=[
            pl.BlockSpec((B, D), lambda i: (0, 0)),
            pl.BlockSpec((B, 1), lambda i: (0, 0)),
            pl.BlockSpec((CB_, D), lambda i: (i, 0)),
        ],
        out_specs=pl.BlockSpec(memory_space=pltpu.SMEM),
        out_shape=jax.ShapeDtypeStruct((1, 1), jnp.float32),
        scratch_shapes=[
            pltpu.VMEM((B, 128), jnp.float32),
            pltpu.VMEM((B, 128), jnp.float32),
        ],
        compiler_params=pltpu.CompilerParams(
            dimension_semantics=("arbitrary",)),
    )(inputs, labels2, tpad)
    return out[0, 0]


# speedup vs baseline: 1.5972x; 1.5972x over previous
import functools

import jax
import jax.numpy as jnp
from jax import lax
from jax.experimental import pallas as pl
from jax.experimental.pallas import tpu as pltpu
from jax.experimental.pallas import tpu_sc as plsc

MARGIN_ = 1.0
EPS_ = 1e-6
CB_ = 512
KAUG_ = 128
NC_, NS_ = 2, 16
NW_ = NC_ * NS_
GRP_ = 2


def _sc_gather_body(table_hbm, idx_hbm, out_hbm, idx_v, idx2_v, rows_v, sem,
                    *, bpw):
    wid = lax.axis_index("s") * NC_ + lax.axis_index("c")
    base = wid * bpw
    pltpu.sync_copy(idx_hbm.at[pl.ds(base, bpw)], idx_v)
    for k in range(bpw // 16):
        sl = pl.ds(k * 16, 16)
        idx2_v[sl] = lax.shift_right_logical(idx_v[sl], 1)
    pltpu.async_copy(table_hbm.at[idx2_v], rows_v, sem).wait()
    pltpu.sync_copy(rows_v, out_hbm.at[pl.ds(base, bpw)])


def _gather_positive_groups(table2, labels):
    B = labels.shape[0]
    W = table2.shape[1]
    bpw = B // NW_
    mesh = plsc.VectorSubcoreMesh(core_axis_name="c", subcore_axis_name="s")
    return pl.kernel(
        functools.partial(_sc_gather_body, bpw=bpw),
        mesh=mesh,
        out_type=jax.ShapeDtypeStruct((B, W), jnp.float32),
        scratch_types=[
            pltpu.VMEM((bpw,), jnp.int32),
            pltpu.VMEM((bpw,), jnp.int32),
            pltpu.VMEM((bpw, W), jnp.float32),
            pltpu.SemaphoreType.DMA,
        ],
    )(table2, labels)


def _dist_body(a_aug_ref, labels_ref, target_ref, minacc_ref, *, n_valid):
    i = pl.program_id(0)
    D = target_ref.shape[1]

    rows = i * CB_ + lax.broadcasted_iota(jnp.int32, (CB_, 1), 0)
    pad = rows >= n_valid
    t = jnp.where(pad, 0.0, target_ref[...] - EPS_)
    t_sq = jnp.sum(t * t, axis=1, keepdims=True)
    t_sq = jnp.where(pad, 3e38, t_sq)
    t_aug = jnp.concatenate(
        [t, t_sq, jnp.zeros((CB_, KAUG_ - D - 1), jnp.float32)], axis=1)

    s = lax.dot_general(a_aug_ref[...], t_aug, (((1,), (1,)), ((), ())),
                        preferred_element_type=jnp.float32)
    cols = i * CB_ + lax.broadcasted_iota(jnp.int32, s.shape, 1)
    s = jnp.where(cols == labels_ref[...], jnp.inf, s)

    @pl.when(i == 0)
    def _init():
        minacc_ref[...] = jnp.full_like(minacc_ref, jnp.inf)

    m = minacc_ref[...]
    for k in range(CB_ // 128):
        m = jnp.minimum(m, s[:, k * 128:(k + 1) * 128])
    minacc_ref[...] = m


def _final_body(minacc_ref, inputs_ref, grp_ref, labels_ref, out_ref):
    a = inputs_ref[...]
    D = a.shape[1]
    sub = jnp.bitwise_and(labels_ref[...], GRP_ - 1)
    pos = jnp.zeros_like(a)
    for k in range(GRP_):
        pos = pos + jnp.where(sub == k, grp_ref[:, k * D:(k + 1) * D], 0.0)
    a_sq = jnp.sum(a * a, axis=1, keepdims=True)
    d_an = jnp.sqrt(jnp.clip(
        a_sq + jnp.min(minacc_ref[...], axis=1, keepdims=True), 1e-12))
    dp = a - pos + EPS_
    d_ap = jnp.sqrt(jnp.clip(jnp.sum(dp * dp, axis=1, keepdims=True), 1e-12))
    per = jnp.maximum(d_ap - d_an + MARGIN_, 0.0)
    out_ref[0, 0] = jnp.sum(per) / a.shape[0]


def kernel(inputs, labels, target):
    B, D = inputs.shape
    C = target.shape[0]
    nblocks = (C + CB_ - 1) // CB_

    grp2 = _gather_positive_groups(target.reshape(C // GRP_, GRP_ * D), labels)

    a_aug = jnp.concatenate(
        [-2.0 * inputs,
         jnp.ones((B, 1), jnp.float32),
         jnp.zeros((B, KAUG_ - D - 1), jnp.float32)], axis=1)
    labels2 = labels.reshape(B, 1)

    minacc = pl.pallas_call(
        functools.partial(_dist_body, n_valid=C),
        grid=(nblocks,),
        in_specs=[
            pl.BlockSpec((B, KAUG_), lambda i: (0, 0)),
            pl.BlockSpec((B, 1), lambda i: (0, 0)),
            pl.BlockSpec((CB_, D), lambda i: (i, 0)),
        ],
        out_specs=pl.BlockSpec((B, 128), lambda i: (0, 0)),
        out_shape=jax.ShapeDtypeStruct((B, 128), jnp.float32),
        compiler_params=pltpu.CompilerParams(
            dimension_semantics=("arbitrary",)),
    )(a_aug, labels2, target)

    out = pl.pallas_call(
        _final_body,
        out_specs=pl.BlockSpec(memory_space=pltpu.SMEM),
        out_shape=jax.ShapeDtypeStruct((1, 1), jnp.float32),
    )(minacc, inputs, grp2, labels2)
    return out[0, 0]


# speedup vs baseline: 1.8685x; 1.1698x over previous
import functools

import jax
import jax.numpy as jnp
from jax import lax
from jax.experimental import pallas as pl
from jax.experimental.pallas import tpu as pltpu

MARGIN_ = 1.0
EPS_ = 1e-6
CB_ = 512
KAUG_ = 128


def _dist_body(a_aug_ref, labels_ref, target_ref, minacc_ref, posacc_ref,
               *, n_valid):
    i = pl.program_id(0)
    D = target_ref.shape[1]

    rows = i * CB_ + lax.broadcasted_iota(jnp.int32, (CB_, 1), 0)
    pad = rows >= n_valid
    t = jnp.where(pad, 0.0, target_ref[...] - EPS_)
    t_sq = jnp.sum(t * t, axis=1, keepdims=True)
    t_sq = jnp.where(pad, 3e38, t_sq)
    t_aug = jnp.concatenate(
        [t, t_sq, jnp.zeros((CB_, KAUG_ - D - 1), jnp.float32)], axis=1)

    s = lax.dot_general(a_aug_ref[...], t_aug, (((1,), (1,)), ((), ())),
                        preferred_element_type=jnp.float32)
    cols = i * CB_ + lax.broadcasted_iota(jnp.int32, s.shape, 1)
    is_own = cols == labels_ref[...]
    s_min = jnp.where(is_own, jnp.inf, s)
    s_pos = jnp.where(is_own, s, 0.0)

    @pl.when(i == 0)
    def _init():
        minacc_ref[...] = jnp.full_like(minacc_ref, jnp.inf)
        posacc_ref[...] = jnp.zeros_like(posacc_ref)

    m = minacc_ref[...]
    p = posacc_ref[...]
    for k in range(CB_ // 128):
        m = jnp.minimum(m, s_min[:, k * 128:(k + 1) * 128])
        p = p + s_pos[:, k * 128:(k + 1) * 128]
    minacc_ref[...] = m
    posacc_ref[...] = p


def _final_body(minacc_ref, posacc_ref, inputs_ref, out_ref):
    a = inputs_ref[...]
    a_sq = jnp.sum(a * a, axis=1, keepdims=True)
    d_an = jnp.sqrt(jnp.clip(
        a_sq + jnp.min(minacc_ref[...], axis=1, keepdims=True), 1e-12))
    d_ap = jnp.sqrt(jnp.clip(
        a_sq + jnp.sum(posacc_ref[...], axis=1, keepdims=True), 1e-12))
    per = jnp.maximum(d_ap - d_an + MARGIN_, 0.0)
    out_ref[0, 0] = jnp.sum(per) / a.shape[0]


def kernel(inputs, labels, target):
    B, D = inputs.shape
    C = target.shape[0]
    nblocks = (C + CB_ - 1) // CB_

    a_aug = jnp.concatenate(
        [-2.0 * inputs,
         jnp.ones((B, 1), jnp.float32),
         jnp.zeros((B, KAUG_ - D - 1), jnp.float32)], axis=1)
    labels2 = labels.reshape(B, 1)

    minacc, posacc = pl.pallas_call(
        functools.partial(_dist_body, n_valid=C),
        grid=(nblocks,),
        in_specs=[
            pl.BlockSpec((B, KAUG_), lambda i: (0, 0)),
            pl.BlockSpec((B, 1), lambda i: (0, 0)),
            pl.BlockSpec((CB_, D), lambda i: (i, 0)),
        ],
        out_specs=[
            pl.BlockSpec((B, 128), lambda i: (0, 0)),
            pl.BlockSpec((B, 128), lambda i: (0, 0)),
        ],
        out_shape=[
            jax.ShapeDtypeStruct((B, 128), jnp.float32),
            jax.ShapeDtypeStruct((B, 128), jnp.float32),
        ],
        compiler_params=pltpu.CompilerParams(
            dimension_semantics=("arbitrary",)),
    )(a_aug, labels2, target)

    out = pl.pallas_call(
        _final_body,
        out_specs=pl.BlockSpec(memory_space=pltpu.SMEM),
        out_shape=jax.ShapeDtypeStruct((1, 1), jnp.float32),
    )(minacc, posacc, inputs)
    return out[0, 0]
